# baseline (speedup 1.0000x reference)
import jax
import jax.numpy as jnp
from jax import lax
from jax.experimental import pallas as pl
from jax.experimental.pallas import tpu as pltpu
from jax.experimental.pallas import tpu_sc as plsc

NC = 2
NS = 16
NW = NC * NS
L = 16

_B, _S, _D = 1024, 200, 64
_ROWS_W = _B // NW


def _make_kernel():
    mesh = plsc.VectorSubcoreMesh(
        core_axis_name="c", subcore_axis_name="s",
        num_cores=NC, num_subcores=NS)

    @pl.kernel(
        out_type=jax.ShapeDtypeStruct((_B, _S, _D), jnp.float32),
        mesh=mesh,
        compiler_params=pltpu.CompilerParams(use_tc_tiling_on_sc=False),
        scratch_types=[
            pltpu.VMEM((_ROWS_W, _S), jnp.int32),
            pltpu.VMEM((_S, _D), jnp.float32),
            pltpu.VMEM((_S, 2 * _D), jnp.float32),
            pltpu.SemaphoreType.DMA,
        ],
    )
    def k(x_hbm, table_hbm, pe_hbm, out_hbm, idx_v, pe_v, rows_v, sem):
        wid = lax.axis_index("s") * NC + lax.axis_index("c")
        base = wid * _ROWS_W
        pltpu.sync_copy(x_hbm.at[pl.ds(base, _ROWS_W)], idx_v)
        pltpu.sync_copy(pe_hbm.at[pl.ds(0, _S)], pe_v)

        def chunk_body(kk, carry):
            pltpu.async_copy(table_hbm.at[idx_v.at[kk]], rows_v, sem).wait()

            def row_body(r, c2):
                for c in range(_D // L):
                    sl = pl.ds(c * L, L)
                    plsc.addupdate(rows_v.at[r, sl], pe_v[r, sl])
                return c2

            lax.fori_loop(0, _S, row_body, 0)
            pltpu.sync_copy(rows_v.at[:, pl.ds(0, _D)], out_hbm.at[base + kk])
            return carry

        lax.fori_loop(0, _ROWS_W, chunk_body, 0)

    return k


_kernel_call = _make_kernel()


def kernel(x, token_table, pe):
    tab128 = jnp.pad(token_table, ((0, 0), (0, _D)))
    return _kernel_call(x, tab128, pe)


# speedup vs baseline: 1.1482x; 1.1482x over previous
import jax
import jax.numpy as jnp
from jax import lax
from jax.experimental import pallas as pl
from jax.experimental.pallas import tpu as pltpu
from jax.experimental.pallas import tpu_sc as plsc

NC = 2
NS = 16
NW = NC * NS
L = 16

_B, _S, _D = 1024, 200, 64
_DP = 2 * _D
_ROWS_W = _B // NW


def _make_kernel():
    mesh = plsc.VectorSubcoreMesh(
        core_axis_name="c", subcore_axis_name="s",
        num_cores=NC, num_subcores=NS)

    @pl.kernel(
        out_type=jax.ShapeDtypeStruct((_B, _S, _D), jnp.float32),
        mesh=mesh,
        compiler_params=pltpu.CompilerParams(use_tc_tiling_on_sc=True),
        scratch_types=[
            pltpu.VMEM((_ROWS_W * _S,), jnp.int32),
            pltpu.VMEM((_S, _D), jnp.float32),
            pltpu.VMEM((_S, _DP), jnp.float32),
            pltpu.VMEM((_S, _D), jnp.float32),
            pltpu.SemaphoreType.DMA,
        ],
    )
    def k(x_hbm, table_hbm, pe_hbm, out_hbm, idx_v, pe_v, rows_v, sum_v, sem):
        wid = lax.axis_index("s") * NC + lax.axis_index("c")
        base = wid * _ROWS_W
        pltpu.sync_copy(x_hbm.at[pl.ds(base * _S, _ROWS_W * _S)], idx_v)
        pltpu.sync_copy(pe_hbm.at[pl.ds(0, _S)], pe_v)

        def chunk_body(kk, carry):
            pltpu.async_copy(
                table_hbm.at[idx_v.at[pl.ds(kk * _S, _S)]], rows_v, sem
            ).wait()

            def row_body(r, c2):
                for c in range(_D // L):
                    sl = pl.ds(c * L, L)
                    sum_v[r, sl] = rows_v[r, sl] + pe_v[r, sl]
                return c2

            lax.fori_loop(0, _S, row_body, 0)
            pltpu.sync_copy(sum_v, out_hbm.at[base + kk])
            return carry

        lax.fori_loop(0, _ROWS_W, chunk_body, 0)

    return k


_kernel_call = _make_kernel()


def kernel(x, token_table, pe):
    tab128 = jnp.pad(token_table, ((0, 0), (0, _D)))
    return _kernel_call(x.reshape(-1), tab128, pe)
